# baseline (speedup 1.0000x reference)
import functools

import jax
import jax.numpy as jnp
from jax.experimental import pallas as pl
from jax.experimental.pallas import tpu as pltpu

_TOKENS = 32768
_DIM = 768
_EXPERTS = 64
_BT = 4096


def _router_kernel(x_ref, c_ref, t_ref, out_ref):
    c = c_ref[:]
    c_ss = jnp.sum(c * c, axis=1, keepdims=True)
    cn = c * jax.lax.rsqrt(jnp.maximum(c_ss, 1e-24))

    xb = x_ref[:]
    x_ss = jnp.sum(xb * xb, axis=1, keepdims=True)
    inv_norm = jax.lax.rsqrt(jnp.maximum(x_ss, 1e-24))

    logits = jax.lax.dot_general(
        xb, cn, (((1,), (1,)), ((), ())), preferred_element_type=jnp.float32
    )
    out_ref[:] = logits * (inv_norm / t_ref[0])


@jax.jit
def kernel(x, centroids, temperature):
    grid = (_TOKENS // _BT,)
    return pl.pallas_call(
        _router_kernel,
        grid=grid,
        in_specs=[
            pl.BlockSpec((_BT, _DIM), lambda i: (i, 0)),
            pl.BlockSpec((_EXPERTS, _DIM), lambda i: (0, 0)),
            pl.BlockSpec(memory_space=pltpu.SMEM),
        ],
        out_specs=pl.BlockSpec((_BT, _EXPERTS), lambda i: (i, 0)),
        out_shape=jax.ShapeDtypeStruct((_TOKENS, _EXPERTS), jnp.float32),
        compiler_params=pltpu.CompilerParams(
            dimension_semantics=("arbitrary",),
        ),
    )(x, centroids, temperature)


# speedup vs baseline: 1.0052x; 1.0052x over previous
import functools

import jax
import jax.numpy as jnp
from jax.experimental import pallas as pl
from jax.experimental.pallas import tpu as pltpu

_TOKENS = 32768
_DIM = 768
_EXPERTS = 64
_BT = 4096


def _router_kernel(x_ref, c_ref, t_ref, out_ref, cn_ref):
    @pl.when(pl.program_id(0) == 0)
    def _init():
        c = c_ref[:]
        c_ss = jnp.sum(c * c, axis=1, keepdims=True)
        cn_ref[:] = c * jax.lax.rsqrt(jnp.maximum(c_ss, 1e-24))

    xb = x_ref[:]
    x_ss = jnp.sum(xb * xb, axis=1, keepdims=True)
    inv_norm = jax.lax.rsqrt(jnp.maximum(x_ss, 1e-24))

    logits = jax.lax.dot_general(
        xb, cn_ref[:], (((1,), (1,)), ((), ())), preferred_element_type=jnp.float32
    )
    out_ref[:] = logits * (inv_norm / t_ref[0])


@jax.jit
def kernel(x, centroids, temperature):
    grid = (_TOKENS // _BT,)
    return pl.pallas_call(
        _router_kernel,
        grid=grid,
        in_specs=[
            pl.BlockSpec((_BT, _DIM), lambda i: (i, 0)),
            pl.BlockSpec((_EXPERTS, _DIM), lambda i: (0, 0)),
            pl.BlockSpec(memory_space=pltpu.SMEM),
        ],
        out_specs=pl.BlockSpec((_BT, _EXPERTS), lambda i: (i, 0)),
        out_shape=jax.ShapeDtypeStruct((_TOKENS, _EXPERTS), jnp.float32),
        scratch_shapes=[pltpu.VMEM((_EXPERTS, _DIM), jnp.float32)],
        compiler_params=pltpu.CompilerParams(
            dimension_semantics=("arbitrary",),
        ),
    )(x, centroids, temperature)


# speedup vs baseline: 1.0412x; 1.0359x over previous
import functools

import jax
import jax.numpy as jnp
from jax.experimental import pallas as pl
from jax.experimental.pallas import tpu as pltpu

_TOKENS = 32768
_DIM = 768
_EXPERTS = 64
_BT = 4096
_STREAMS = 4
_HB = _BT // _STREAMS


def _router_kernel(x0_ref, x1_ref, x2_ref, x3_ref, c_ref, t_ref, out_ref, cn_ref):
    @pl.when(pl.program_id(0) == 0)
    def _init():
        c = c_ref[:]
        c_ss = jnp.sum(c * c, axis=1, keepdims=True)
        cn_ref[:] = c * jax.lax.rsqrt(jnp.maximum(c_ss, 1e-24))

    inv_t = 1.0 / t_ref[0]
    cn = cn_ref[:]
    for k, x_ref in enumerate((x0_ref, x1_ref, x2_ref, x3_ref)):
        xb = x_ref[:]
        x_ss = jnp.sum(xb * xb, axis=1, keepdims=True)
        inv_norm = jax.lax.rsqrt(jnp.maximum(x_ss, 1e-24))
        logits = jax.lax.dot_general(
            xb, cn, (((1,), (1,)), ((), ())), preferred_element_type=jnp.float32
        )
        out_ref[k * _HB:(k + 1) * _HB, :] = logits * (inv_norm * inv_t)


@jax.jit
def kernel(x, centroids, temperature):
    grid = (_TOKENS // _BT,)
    x_specs = [
        pl.BlockSpec((_HB, _DIM), functools.partial(lambda k, i: (_STREAMS * i + k, 0), k))
        for k in range(_STREAMS)
    ]
    return pl.pallas_call(
        _router_kernel,
        grid=grid,
        in_specs=x_specs + [
            pl.BlockSpec((_EXPERTS, _DIM), lambda i: (0, 0)),
            pl.BlockSpec(memory_space=pltpu.SMEM),
        ],
        out_specs=pl.BlockSpec((_BT, _EXPERTS), lambda i: (i, 0)),
        out_shape=jax.ShapeDtypeStruct((_TOKENS, _EXPERTS), jnp.float32),
        scratch_shapes=[pltpu.VMEM((_EXPERTS, _DIM), jnp.float32)],
        compiler_params=pltpu.CompilerParams(
            dimension_semantics=("arbitrary",),
        ),
    )(x, x, x, x, centroids, temperature)
